# baseline (speedup 1.0000x reference)
import functools

import jax
import jax.numpy as jnp
from jax import lax
from jax.experimental import pallas as pl
from jax.experimental.pallas import tpu as pltpu
from jax.experimental.pallas import tpu_sc as plsc

_NC = 2
_NS = 16
_NW = _NC * _NS
_CH = 128


def _sc_gather(table, idx3d, n_rows, d):
    chunks_per_w = n_rows // (_NW * _CH)
    mesh = plsc.VectorSubcoreMesh(core_axis_name="c", subcore_axis_name="s")

    @functools.partial(
        pl.kernel,
        out_type=jax.ShapeDtypeStruct((n_rows, d), jnp.float32),
        mesh=mesh,
        scratch_types=[
            pltpu.VMEM((chunks_per_w, _CH), jnp.int32),
            pltpu.VMEM((_CH, d), jnp.float32),
            pltpu.VMEM((_CH, d), jnp.float32),
            pltpu.SemaphoreType.DMA,
            pltpu.SemaphoreType.DMA,
        ],
    )
    def k(table_hbm, idx_hbm, out_hbm, idx_v, buf0, buf1, sem0, sem1):
        wid = lax.axis_index("s") * _NC + lax.axis_index("c")
        row0 = wid * chunks_per_w * _CH
        pltpu.sync_copy(idx_hbm.at[wid], idx_v)

        def fire(g, buf, sem):
            pltpu.make_async_copy(table_hbm.at[idx_v.at[g]], buf, sem).start()

        def drain(g, buf, sem):
            pltpu.make_async_copy(table_hbm.at[idx_v.at[g]], buf, sem).wait()
            pltpu.sync_copy(buf, out_hbm.at[pl.ds(row0 + g * _CH, _CH)])

        fire(0, buf0, sem0)

        def step(i, cur, csem, nxt, nsem):
            @pl.when(i + 1 < chunks_per_w)
            def _():
                fire(i + 1, nxt, nsem)

            drain(i, cur, csem)

        def body(i, carry):
            @pl.when(i % 2 == 0)
            def _():
                step(i, buf0, sem0, buf1, sem1)

            @pl.when(i % 2 == 1)
            def _():
                step(i, buf1, sem1, buf0, sem0)

            return carry

        lax.fori_loop(0, chunks_per_w, body, 0)

    return k(table, idx3d)


def _ctx_body(h_ref, gn_ref, w1_ref, b1_ref, w2_ref, b2_ref, gm_ref, bt_ref,
              wg_ref, wr_ref, br_ref, ctx_ref, *, bt, t):
    d = h_ref.shape[-1]
    tm1 = t - 1
    h2 = h_ref[...]
    z = jnp.maximum(
        jnp.dot(h2, w1_ref[...], preferred_element_type=jnp.float32)
        + b1_ref[...], 0.0)
    ff = jnp.dot(z, w2_ref[...], preferred_element_type=jnp.float32) + b2_ref[...]
    x = h2 + ff
    mu = jnp.mean(x, axis=1, keepdims=True)
    var = jnp.mean((x - mu) ** 2, axis=1, keepdims=True)
    hid2 = (x - mu) * lax.rsqrt(var + 1e-05) * gm_ref[...] + bt_ref[...]

    dn_rc = (((1,), (1,)), ((), ()))
    dn_maj = (((0,), (0,)), ((), ()))
    gate_col = lax.dot_general(hid2, wg_ref[...], dn_rc,
                               preferred_element_type=jnp.float32)
    r_seg = lax.broadcasted_iota(jnp.int32, (bt * t, t), 0) // t
    r_pos = lax.broadcasted_iota(jnp.int32, (bt * t, t), 0) % t
    col_t = lax.broadcasted_iota(jnp.int32, (bt * t, t), 1)
    oh = (r_pos == col_t).astype(jnp.float32)
    e_cols = lax.broadcasted_iota(jnp.int32, (bt * t, bt), 1)
    emat = (r_seg[:, :bt] == e_cols).astype(jnp.float32)
    gate = lax.dot_general(emat, gate_col * oh, dn_maj,
                           preferred_element_type=jnp.float32)
    iota = lax.broadcasted_iota(jnp.int32, (bt, t), 1)
    neg = jnp.float32(-3e38)
    gate = jnp.where(iota == tm1, neg, gate)
    pert = gate + gn_ref[...]

    big = jnp.int32(t + 1)

    def top3(scores):
        masks = []
        xm = scores
        for _ in range(3):
            v = jnp.max(xm, axis=1, keepdims=True)
            first = jnp.min(jnp.where(xm == v, iota, big), axis=1, keepdims=True)
            m = iota == first
            masks.append(m)
            xm = jnp.where(m, neg, xm)
        return masks

    pm = top3(pert)
    pmask = pm[0] | pm[1] | pm[2]
    gm3 = top3(gate)
    mrows = [g.astype(jnp.float32) for g in gm3]
    ohlast = (lax.broadcasted_iota(jnp.int32, (1, t), 1) == tm1).astype(
        jnp.float32)

    mrows.append(jnp.broadcast_to(ohlast, (bt, t)))
    mtiled = jnp.concatenate(
        [jnp.concatenate([m] * bt, axis=1) for m in mrows], axis=0)
    sr_b = lax.broadcasted_iota(jnp.int32, (4 * bt, bt * t), 0) % bt
    sc_b = lax.broadcasted_iota(jnp.int32, (4 * bt, bt * t), 1) // t
    mbig = jnp.where(sr_b == sc_b, mtiled, 0.0)
    selmat = jnp.dot(mbig, hid2, preferred_element_type=jnp.float32)
    cs = [selmat[j * bt:(j + 1) * bt] for j in range(3)]
    qin = selmat[3 * bt:4 * bt]

    q = jnp.dot(qin, wr_ref[...], preferred_element_type=jnp.float32) + br_ref[...]

    ws, ss = [], []
    for j in range(3):
        w_j = jnp.sum(jnp.where(gm3[j] & pmask, 1.0, 0.0), axis=1, keepdims=True)
        s_j = w_j * jnp.sum(cs[j] * q, axis=1, keepdims=True)
        ws.append(w_j)
        ss.append(s_j)

    smax = jnp.maximum(jnp.maximum(ss[0], ss[1]), jnp.maximum(ss[2], 0.0))
    es = [jnp.exp(s - smax) for s in ss]
    zden = 61.0 * jnp.exp(-smax) + es[0] + es[1] + es[2]
    ctx = sum((e * w / zden) * c for e, w, c in zip(es, ws, cs))
    ctx_ref[...] = ctx


def _ctx_stage(h_flat, gn_pad, b, t, W1, b1, W2, b2, gamma, beta, wg_row, Wr,
               br_row):
    d = h_flat.shape[-1]
    bt = 64
    grid = b // bt
    body = functools.partial(_ctx_body, bt=bt, t=t)
    return pl.pallas_call(
        body,
        grid=(grid,),
        in_specs=[
            pl.BlockSpec((bt * t, d), lambda i: (i, 0)),
            pl.BlockSpec((bt, t), lambda i: (i, 0)),
            pl.BlockSpec(W1.shape, lambda i: (0, 0)),
            pl.BlockSpec((1, 2 * d), lambda i: (0, 0)),
            pl.BlockSpec(W2.shape, lambda i: (0, 0)),
            pl.BlockSpec((1, d), lambda i: (0, 0)),
            pl.BlockSpec((1, d), lambda i: (0, 0)),
            pl.BlockSpec((1, d), lambda i: (0, 0)),
            pl.BlockSpec((1, d), lambda i: (0, 0)),
            pl.BlockSpec(Wr.shape, lambda i: (0, 0)),
            pl.BlockSpec((1, d), lambda i: (0, 0)),
        ],
        out_specs=pl.BlockSpec((bt, d), lambda i: (i, 0)),
        out_shape=jax.ShapeDtypeStruct((b, d), jnp.float32),
    )(h_flat, gn_pad, W1, b1.reshape(1, -1), W2, b2.reshape(1, -1),
      gamma.reshape(1, -1), beta.reshape(1, -1), wg_row, Wr, br_row)


def _proj_body(ctx_ref, wo_ref, bo_ref, out_ref):
    out_ref[...] = (
        jnp.dot(ctx_ref[...], wo_ref[...], preferred_element_type=jnp.float32)
        + bo_ref[...])


def _proj_stage(ctx, Wo, bo_row):
    b, d = ctx.shape
    v = Wo.shape[1]
    vt = 4096
    return pl.pallas_call(
        _proj_body,
        grid=(pl.cdiv(v, vt),),
        in_specs=[
            pl.BlockSpec((b, d), lambda i: (0, 0)),
            pl.BlockSpec((d, vt), lambda i: (0, i)),
            pl.BlockSpec((1, vt), lambda i: (0, i)),
        ],
        out_specs=pl.BlockSpec((b, vt), lambda i: (0, i)),
        out_shape=jax.ShapeDtypeStruct((b, v), jnp.float32),
    )(ctx, Wo, bo_row)


def kernel(seq, temperature, embed, W1, b1, W2, b2, gamma, beta, Wg, bg, Wr,
           br, Wo, bo):
    b, t = seq.shape
    v, d = embed.shape
    half = b // 2
    rows_h = half * t

    u = jax.random.uniform(jax.random.key(42), (b, t - 1))
    gnoise = -jnp.log(-jnp.log(u + 1e-08) + 1e-08)
    gn_pad = jnp.pad(gnoise, ((0, 0), (0, 1)))

    seq32 = seq.astype(jnp.int32)
    wg_row = Wg.reshape(1, d)
    br_row = br.reshape(1, d)

    h_halves = []
    for i in range(2):
        idx3d = seq32[i * half:(i + 1) * half].reshape(
            _NW, rows_h // (_NW * _CH), _CH)
        h_halves.append(_sc_gather(embed, idx3d, rows_h, d))
    ctx_halves = [
        _ctx_stage(h_halves[i], gn_pad[i * half:(i + 1) * half], half, t,
                   W1, b1, W2, b2, gamma, beta, wg_row, Wr, br_row)
        for i in range(2)
    ]
    ctx = jnp.concatenate(ctx_halves, axis=0)
    return _proj_stage(ctx, Wo, bo.reshape(1, -1))


# speedup vs baseline: 1.1620x; 1.1620x over previous
import functools

import jax
import jax.numpy as jnp
from jax import lax
from jax.experimental import pallas as pl
from jax.experimental.pallas import tpu as pltpu
from jax.experimental.pallas import tpu_sc as plsc

_NC = 2
_NS = 16
_NW = _NC * _NS
_CH = 128


def _sc_gather(table, idx3d, n_rows, d):
    chunks_per_w = n_rows // (_NW * _CH)
    mesh = plsc.VectorSubcoreMesh(core_axis_name="c", subcore_axis_name="s")

    @functools.partial(
        pl.kernel,
        out_type=jax.ShapeDtypeStruct((n_rows, d), jnp.float32),
        mesh=mesh,
        scratch_types=[
            pltpu.VMEM((chunks_per_w, _CH), jnp.int32),
            pltpu.VMEM((_CH, d), jnp.float32),
            pltpu.VMEM((_CH, d), jnp.float32),
            pltpu.SemaphoreType.DMA,
            pltpu.SemaphoreType.DMA,
        ],
    )
    def k(table_hbm, idx_hbm, out_hbm, idx_v, buf0, buf1, sem0, sem1):
        wid = lax.axis_index("s") * _NC + lax.axis_index("c")
        row0 = wid * chunks_per_w * _CH
        pltpu.sync_copy(idx_hbm.at[wid], idx_v)

        def fire(g, buf, sem):
            pltpu.make_async_copy(table_hbm.at[idx_v.at[g]], buf, sem).start()

        def drain(g, buf, sem):
            pltpu.make_async_copy(table_hbm.at[idx_v.at[g]], buf, sem).wait()
            pltpu.sync_copy(buf, out_hbm.at[pl.ds(row0 + g * _CH, _CH)])

        fire(0, buf0, sem0)

        def step(i, cur, csem, nxt, nsem):
            @pl.when(i + 1 < chunks_per_w)
            def _():
                fire(i + 1, nxt, nsem)

            drain(i, cur, csem)

        def body(i, carry):
            @pl.when(i % 2 == 0)
            def _():
                step(i, buf0, sem0, buf1, sem1)

            @pl.when(i % 2 == 1)
            def _():
                step(i, buf1, sem1, buf0, sem0)

            return carry

        lax.fori_loop(0, chunks_per_w, body, 0)

    return k(table, idx3d)


def _ctx_body(h_ref, gn_ref, w1_ref, b1_ref, w2_ref, b2_ref, gm_ref, bt_ref,
              wg_ref, wr_ref, br_ref, ctx_ref, *, bt, t):
    d = h_ref.shape[-1]
    tm1 = t - 1
    h2 = h_ref[...]
    z = jnp.maximum(
        jnp.dot(h2, w1_ref[...], preferred_element_type=jnp.float32)
        + b1_ref[...], 0.0)
    ff = jnp.dot(z, w2_ref[...], preferred_element_type=jnp.float32) + b2_ref[...]
    x = h2 + ff
    mu = jnp.mean(x, axis=1, keepdims=True)
    var = jnp.mean((x - mu) ** 2, axis=1, keepdims=True)
    hid2 = (x - mu) * lax.rsqrt(var + 1e-05) * gm_ref[...] + bt_ref[...]

    segs = [hid2[b * t:(b + 1) * t, :] for b in range(bt)]
    dn_rc = (((1,), (1,)), ((), ()))
    gate = jnp.concatenate(
        [lax.dot_general(wg_ref[...], segs[b], dn_rc,
                         preferred_element_type=jnp.float32) for b in range(bt)],
        axis=0)
    iota = lax.broadcasted_iota(jnp.int32, (bt, t), 1)
    neg = jnp.float32(-3e38)
    gate = jnp.where(iota == tm1, neg, gate)
    pert = gate + gn_ref[...]

    big = jnp.int32(t + 1)

    def top3(scores):
        masks = []
        xm = scores
        for _ in range(3):
            v = jnp.max(xm, axis=1, keepdims=True)
            first = jnp.min(jnp.where(xm == v, iota, big), axis=1, keepdims=True)
            m = iota == first
            masks.append(m)
            xm = jnp.where(m, neg, xm)
        return masks

    pm = top3(pert)
    pmask = pm[0] | pm[1] | pm[2]
    gm3 = top3(gate)
    mrows = [g.astype(jnp.float32) for g in gm3]
    ohlast = (lax.broadcasted_iota(jnp.int32, (1, t), 1) == tm1).astype(
        jnp.float32)

    sel = []
    for b in range(bt):
        mb = jnp.concatenate(
            [mrows[0][b:b + 1], mrows[1][b:b + 1], mrows[2][b:b + 1], ohlast],
            axis=0)
        sel.append(jnp.dot(mb, segs[b], preferred_element_type=jnp.float32))
    cs = [jnp.concatenate([s[j:j + 1] for s in sel], axis=0) for j in range(3)]
    qin = jnp.concatenate([s[3:4] for s in sel], axis=0)

    q = jnp.dot(qin, wr_ref[...], preferred_element_type=jnp.float32) + br_ref[...]

    ws, ss = [], []
    for j in range(3):
        w_j = jnp.sum(jnp.where(gm3[j] & pmask, 1.0, 0.0), axis=1, keepdims=True)
        s_j = w_j * jnp.sum(cs[j] * q, axis=1, keepdims=True)
        ws.append(w_j)
        ss.append(s_j)

    smax = jnp.maximum(jnp.maximum(ss[0], ss[1]), jnp.maximum(ss[2], 0.0))
    es = [jnp.exp(s - smax) for s in ss]
    zden = 61.0 * jnp.exp(-smax) + es[0] + es[1] + es[2]
    ctx = sum((e * w / zden) * c for e, w, c in zip(es, ws, cs))
    ctx_ref[...] = ctx


def _ctx_stage(h_flat, gn_pad, b, t, W1, b1, W2, b2, gamma, beta, wg_row, Wr,
               br_row):
    d = h_flat.shape[-1]
    bt = 64
    grid = b // bt
    body = functools.partial(_ctx_body, bt=bt, t=t)
    return pl.pallas_call(
        body,
        grid=(grid,),
        in_specs=[
            pl.BlockSpec((bt * t, d), lambda i: (i, 0)),
            pl.BlockSpec((bt, t), lambda i: (i, 0)),
            pl.BlockSpec(W1.shape, lambda i: (0, 0)),
            pl.BlockSpec((1, 2 * d), lambda i: (0, 0)),
            pl.BlockSpec(W2.shape, lambda i: (0, 0)),
            pl.BlockSpec((1, d), lambda i: (0, 0)),
            pl.BlockSpec((1, d), lambda i: (0, 0)),
            pl.BlockSpec((1, d), lambda i: (0, 0)),
            pl.BlockSpec((1, d), lambda i: (0, 0)),
            pl.BlockSpec(Wr.shape, lambda i: (0, 0)),
            pl.BlockSpec((1, d), lambda i: (0, 0)),
        ],
        out_specs=pl.BlockSpec((bt, d), lambda i: (i, 0)),
        out_shape=jax.ShapeDtypeStruct((b, d), jnp.float32),
    )(h_flat, gn_pad, W1, b1.reshape(1, -1), W2, b2.reshape(1, -1),
      gamma.reshape(1, -1), beta.reshape(1, -1), wg_row, Wr, br_row)


def _proj_body(ctx_ref, wo_ref, bo_ref, out_ref):
    out_ref[...] = (
        jnp.dot(ctx_ref[...], wo_ref[...], preferred_element_type=jnp.float32)
        + bo_ref[...])


def _proj_stage(ctx, Wo, bo_row):
    b, d = ctx.shape
    v = Wo.shape[1]
    vt = 4096
    return pl.pallas_call(
        _proj_body,
        grid=(pl.cdiv(v, vt),),
        in_specs=[
            pl.BlockSpec((b, d), lambda i: (0, 0)),
            pl.BlockSpec((d, vt), lambda i: (0, i)),
            pl.BlockSpec((1, vt), lambda i: (0, i)),
        ],
        out_specs=pl.BlockSpec((b, vt), lambda i: (0, i)),
        out_shape=jax.ShapeDtypeStruct((b, v), jnp.float32),
    )(ctx, Wo, bo_row)


def kernel(seq, temperature, embed, W1, b1, W2, b2, gamma, beta, Wg, bg, Wr,
           br, Wo, bo):
    b, t = seq.shape
    v, d = embed.shape
    half = b // 2
    rows_h = half * t

    u = jax.random.uniform(jax.random.key(42), (b, t - 1))
    gnoise = -jnp.log(-jnp.log(u + 1e-08) + 1e-08)
    gn_pad = jnp.pad(gnoise, ((0, 0), (0, 1)))

    seq32 = seq.astype(jnp.int32)
    wg_row = Wg.reshape(1, d)
    br_row = br.reshape(1, d)

    h_halves = []
    for i in range(2):
        idx3d = seq32[i * half:(i + 1) * half].reshape(
            _NW, rows_h // (_NW * _CH), _CH)
        h_halves.append(_sc_gather(embed, idx3d, rows_h, d))
    ctx_halves = [
        _ctx_stage(h_halves[i], gn_pad[i * half:(i + 1) * half], half, t,
                   W1, b1, W2, b2, gamma, beta, wg_row, Wr, br_row)
        for i in range(2)
    ]
    ctx = jnp.concatenate(ctx_halves, axis=0)
    return _proj_stage(ctx, Wo, bo.reshape(1, -1))


# speedup vs baseline: 1.1662x; 1.0037x over previous
import functools

import jax
import jax.numpy as jnp
from jax import lax
from jax.experimental import pallas as pl
from jax.experimental.pallas import tpu as pltpu
from jax.experimental.pallas import tpu_sc as plsc

_NC = 2
_NS = 16
_NW = _NC * _NS
_CH = 128


def _sc_gather(table, idx3d, n_rows, d):
    chunks_per_w = n_rows // (_NW * _CH)
    mesh = plsc.VectorSubcoreMesh(core_axis_name="c", subcore_axis_name="s")

    @functools.partial(
        pl.kernel,
        out_type=jax.ShapeDtypeStruct((n_rows, d), jnp.float32),
        mesh=mesh,
        scratch_types=[
            pltpu.VMEM((chunks_per_w, _CH), jnp.int32),
            pltpu.VMEM((_CH, d), jnp.float32),
            pltpu.VMEM((_CH, d), jnp.float32),
            pltpu.SemaphoreType.DMA,
            pltpu.SemaphoreType.DMA,
        ],
    )
    def k(table_hbm, idx_hbm, out_hbm, idx_v, buf0, buf1, sem0, sem1):
        wid = lax.axis_index("s") * _NC + lax.axis_index("c")
        row0 = wid * chunks_per_w * _CH
        pltpu.sync_copy(idx_hbm.at[wid], idx_v)

        def fire(g, buf, sem):
            pltpu.make_async_copy(table_hbm.at[idx_v.at[g]], buf, sem).start()

        def drain(g, buf, sem):
            pltpu.make_async_copy(table_hbm.at[idx_v.at[g]], buf, sem).wait()
            pltpu.sync_copy(buf, out_hbm.at[pl.ds(row0 + g * _CH, _CH)])

        fire(0, buf0, sem0)

        def step(i, cur, csem, nxt, nsem):
            @pl.when(i + 1 < chunks_per_w)
            def _():
                fire(i + 1, nxt, nsem)

            drain(i, cur, csem)

        def body(i, carry):
            @pl.when(i % 2 == 0)
            def _():
                step(i, buf0, sem0, buf1, sem1)

            @pl.when(i % 2 == 1)
            def _():
                step(i, buf1, sem1, buf0, sem0)

            return carry

        lax.fori_loop(0, chunks_per_w, body, 0)

    return k(table, idx3d)


def _ctx_body(h_ref, gn_ref, w1_ref, b1_ref, w2_ref, b2_ref, gm_ref, bt_ref,
              wg_ref, wr_ref, br_ref, ctx_ref, *, bt, t):
    d = h_ref.shape[-1]
    tm1 = t - 1
    h2 = h_ref[...]
    z = jnp.maximum(
        jnp.dot(h2, w1_ref[...], preferred_element_type=jnp.float32)
        + b1_ref[...], 0.0)
    ff = jnp.dot(z, w2_ref[...], preferred_element_type=jnp.float32) + b2_ref[...]
    x = h2 + ff
    mu = jnp.mean(x, axis=1, keepdims=True)
    var = jnp.mean((x - mu) ** 2, axis=1, keepdims=True)
    hid2 = (x - mu) * lax.rsqrt(var + 1e-05) * gm_ref[...] + bt_ref[...]

    segs = [hid2[b * t:(b + 1) * t, :] for b in range(bt)]
    dn_rc = (((1,), (1,)), ((), ()))
    gate = jnp.concatenate(
        [lax.dot_general(wg_ref[...], segs[b], dn_rc,
                         preferred_element_type=jnp.float32) for b in range(bt)],
        axis=0)
    iota = lax.broadcasted_iota(jnp.int32, (bt, t), 1)
    neg = jnp.float32(-3e38)
    gate = jnp.where(iota == tm1, neg, gate)
    pert = gate + gn_ref[...]

    big = jnp.int32(t + 1)

    def top3(scores):
        masks = []
        xm = scores
        for _ in range(3):
            v = jnp.max(xm, axis=1, keepdims=True)
            first = jnp.min(jnp.where(xm == v, iota, big), axis=1, keepdims=True)
            m = iota == first
            masks.append(m)
            xm = jnp.where(m, neg, xm)
        return masks

    pm = top3(pert)
    pmask = pm[0] | pm[1] | pm[2]
    gm3 = top3(gate)
    mrows = [g.astype(jnp.float32) for g in gm3]
    ohlast = (lax.broadcasted_iota(jnp.int32, (1, t), 1) == tm1).astype(
        jnp.float32)

    sel = []
    for b in range(bt):
        mb = jnp.concatenate(
            [mrows[0][b:b + 1], mrows[1][b:b + 1], mrows[2][b:b + 1], ohlast],
            axis=0)
        sel.append(jnp.dot(mb, segs[b], preferred_element_type=jnp.float32))
    cs = [jnp.concatenate([s[j:j + 1] for s in sel], axis=0) for j in range(3)]
    qin = jnp.concatenate([s[3:4] for s in sel], axis=0)

    q = jnp.dot(qin, wr_ref[...], preferred_element_type=jnp.float32) + br_ref[...]

    ws, ss = [], []
    for j in range(3):
        w_j = jnp.sum(jnp.where(gm3[j] & pmask, 1.0, 0.0), axis=1, keepdims=True)
        s_j = w_j * jnp.sum(cs[j] * q, axis=1, keepdims=True)
        ws.append(w_j)
        ss.append(s_j)

    smax = jnp.maximum(jnp.maximum(ss[0], ss[1]), jnp.maximum(ss[2], 0.0))
    es = [jnp.exp(s - smax) for s in ss]
    zden = 61.0 * jnp.exp(-smax) + es[0] + es[1] + es[2]
    ctx = sum((e * w / zden) * c for e, w, c in zip(es, ws, cs))
    ctx_ref[...] = ctx


def _ctx_stage(h_flat, gn_pad, b, t, W1, b1, W2, b2, gamma, beta, wg_row, Wr,
               br_row):
    d = h_flat.shape[-1]
    bt = 128
    grid = b // bt
    body = functools.partial(_ctx_body, bt=bt, t=t)
    return pl.pallas_call(
        body,
        grid=(grid,),
        in_specs=[
            pl.BlockSpec((bt * t, d), lambda i: (i, 0)),
            pl.BlockSpec((bt, t), lambda i: (i, 0)),
            pl.BlockSpec(W1.shape, lambda i: (0, 0)),
            pl.BlockSpec((1, 2 * d), lambda i: (0, 0)),
            pl.BlockSpec(W2.shape, lambda i: (0, 0)),
            pl.BlockSpec((1, d), lambda i: (0, 0)),
            pl.BlockSpec((1, d), lambda i: (0, 0)),
            pl.BlockSpec((1, d), lambda i: (0, 0)),
            pl.BlockSpec((1, d), lambda i: (0, 0)),
            pl.BlockSpec(Wr.shape, lambda i: (0, 0)),
            pl.BlockSpec((1, d), lambda i: (0, 0)),
        ],
        out_specs=pl.BlockSpec((bt, d), lambda i: (i, 0)),
        out_shape=jax.ShapeDtypeStruct((b, d), jnp.float32),
    )(h_flat, gn_pad, W1, b1.reshape(1, -1), W2, b2.reshape(1, -1),
      gamma.reshape(1, -1), beta.reshape(1, -1), wg_row, Wr, br_row)


def _proj_body(ctx_ref, wo_ref, bo_ref, out_ref):
    out_ref[...] = (
        jnp.dot(ctx_ref[...], wo_ref[...], preferred_element_type=jnp.float32)
        + bo_ref[...])


def _proj_stage(ctx, Wo, bo_row):
    b, d = ctx.shape
    v = Wo.shape[1]
    vt = 4096
    return pl.pallas_call(
        _proj_body,
        grid=(pl.cdiv(v, vt),),
        in_specs=[
            pl.BlockSpec((b, d), lambda i: (0, 0)),
            pl.BlockSpec((d, vt), lambda i: (0, i)),
            pl.BlockSpec((1, vt), lambda i: (0, i)),
        ],
        out_specs=pl.BlockSpec((b, vt), lambda i: (0, i)),
        out_shape=jax.ShapeDtypeStruct((b, v), jnp.float32),
    )(ctx, Wo, bo_row)


def kernel(seq, temperature, embed, W1, b1, W2, b2, gamma, beta, Wg, bg, Wr,
           br, Wo, bo):
    b, t = seq.shape
    v, d = embed.shape
    half = b // 2
    rows_h = half * t

    u = jax.random.uniform(jax.random.key(42), (b, t - 1))
    gnoise = -jnp.log(-jnp.log(u + 1e-08) + 1e-08)
    gn_pad = jnp.pad(gnoise, ((0, 0), (0, 1)))

    seq32 = seq.astype(jnp.int32)
    wg_row = Wg.reshape(1, d)
    br_row = br.reshape(1, d)

    h_halves = []
    for i in range(2):
        idx3d = seq32[i * half:(i + 1) * half].reshape(
            _NW, rows_h // (_NW * _CH), _CH)
        h_halves.append(_sc_gather(embed, idx3d, rows_h, d))
    ctx_halves = [
        _ctx_stage(h_halves[i], gn_pad[i * half:(i + 1) * half], half, t,
                   W1, b1, W2, b2, gamma, beta, wg_row, Wr, br_row)
        for i in range(2)
    ]
    ctx = jnp.concatenate(ctx_halves, axis=0)
    return _proj_stage(ctx, Wo, bo.reshape(1, -1))
